# Initial kernel scaffold; baseline (speedup 1.0000x reference)
import jax
import jax.numpy as jnp
from jax.experimental import pallas as pl


def kernel(edge_index, edge_attr, movie_w, user_w, W1_self, W1_neigh, b1, W2_self, W2_neigh, b2):
    raise NotImplementedError("write your pallas kernel here")


# speedup vs baseline: 3.2858x; 3.2858x over previous
import functools

import jax
import jax.numpy as jnp
from jax import lax
from jax.experimental import pallas as pl
from jax.experimental.pallas import tpu as pltpu
from jax.experimental.pallas import tpu_sc as plsc

N = 50000
E = 800000
D = 64
NC, NS = 2, 16
HALF = N // 2
TROWS = 1568
SROWS = NS * TROWS
DUMMY = 25080
CPE = 128
NCHUNK = E // CPE
LTROWS = HALF - (NS - 1) * TROWS

_MESH = plsc.VectorSubcoreMesh(core_axis_name="c", subcore_axis_name="s")


def _tc_proj(operands, w_neigh, w_self, b2d, do_relu):
    R = 2000
    with_agg = len(operands) == 3

    def body(*refs):
        if with_agg:
            z_ref, a_ref, d_ref = refs[0:3]
            wn_ref, ws_ref, b_ref, y_ref, zo_ref = refs[3:]
            inv = 1.0 / jnp.maximum(d_ref[...][:, 0:1], 1.0)
            t = z_ref[...] + a_ref[...] * inv
        else:
            (z_ref, wn_ref, ws_ref, b_ref, y_ref, zo_ref) = refs
            t = z_ref[...]
        if do_relu:
            t = jnp.maximum(t, 0.0)
        y_ref[...] = jnp.dot(t, wn_ref[...], preferred_element_type=jnp.float32)
        zo_ref[...] = (
            jnp.dot(t, ws_ref[...], preferred_element_type=jnp.float32) + b_ref[...]
        )

    row_specs = [pl.BlockSpec((R, D), lambda i: (i, 0))] * (2 if with_agg else 1)
    if with_agg:
        row_specs.append(pl.BlockSpec((R, 16), lambda i: (i, 0)))
    return pl.pallas_call(
        body,
        grid=(N // R,),
        in_specs=row_specs + [
            pl.BlockSpec((D, D), lambda i: (0, 0)),
            pl.BlockSpec((D, D), lambda i: (0, 0)),
            pl.BlockSpec((1, D), lambda i: (0, 0)),
        ],
        out_specs=[
            pl.BlockSpec((R, D), lambda i: (i, 0)),
            pl.BlockSpec((R, D), lambda i: (i, 0)),
        ],
        out_shape=[
            jax.ShapeDtypeStruct((N, D), jnp.float32),
            jax.ShapeDtypeStruct((N, D), jnp.float32),
        ],
    )(*operands, w_neigh, w_self, b2d)


def _tc_combine(z, agg, deg):
    R = 2000

    def body(z_ref, a_ref, d_ref, o_ref):
        inv = 1.0 / jnp.maximum(d_ref[...][:, 0:1], 1.0)
        o_ref[...] = z_ref[...] + a_ref[...] * inv

    return pl.pallas_call(
        body,
        grid=(N // R,),
        in_specs=[
            pl.BlockSpec((R, D), lambda i: (i, 0)),
            pl.BlockSpec((R, D), lambda i: (i, 0)),
            pl.BlockSpec((R, 16), lambda i: (i, 0)),
        ],
        out_specs=pl.BlockSpec((R, D), lambda i: (i, 0)),
        out_shape=jax.ShapeDtypeStruct((N, D), jnp.float32),
    )(z, agg, deg)


@functools.partial(
    pl.kernel,
    out_type=jax.ShapeDtypeStruct((N, 16), jnp.float32),
    mesh=_MESH,
    compiler_params=pltpu.CompilerParams(use_tc_tiling_on_sc=False, needs_layout_passes=False),
    scratch_types=[
        pltpu.VMEM((CPE,), jnp.int32),
        pltpu.VMEM((CPE,), jnp.int32),
        pltpu.VMEM((CPE, 16), jnp.float32),
        pltpu.VMEM((224, 16), jnp.float32),
        pltpu.VMEM_SHARED((SROWS, 16), jnp.float32),
        pltpu.SemaphoreType.DMA,
        pltpu.SemaphoreType.DMA,
    ],
)
def _deg(dst_hbm, deg_out, di0, di1, onesb, zdbuf, deg_sh, ds0, ds1):
    c = lax.axis_index("c")
    s = lax.axis_index("s")
    lo = c * HALF
    start = s * 390 + jnp.minimum(s, 10)
    cnt = jnp.where(s < 10, 391, 390)
    slots = ((di0, ds0), (di1, ds1))
    z16 = jnp.zeros((16,), jnp.float32)
    ones16 = jnp.ones((16,), jnp.float32)
    l0 = pl.multiple_of(s * TROWS, 8)

    def _zb(r, _):
        zdbuf[r, pl.ds(0, 16)] = z16
        return 0
    lax.fori_loop(0, 224, _zb, 0)

    def _ob(r, _):
        onesb[r, pl.ds(0, 16)] = ones16
        return 0
    lax.fori_loop(0, CPE, _ob, 0)

    for q in range(TROWS // 224):
        pltpu.sync_copy(zdbuf, deg_sh.at[pl.ds(l0 + q * 224, 224)])
    plsc.subcore_barrier()

    def issue_idx(k, slot):
        base = pl.multiple_of((start + k) * CPE, 8)
        pltpu.async_copy(dst_hbm.at[pl.ds(base, CPE)], slot[0], slot[1])

    def wait_idx(slot):
        pltpu.make_async_copy(dst_hbm.at[pl.ds(0, CPE)], slot[0], slot[1]).wait()

    def mask_dst(di):
        for v in range(CPE // 16):
            d = di[pl.ds(v * 16, 16)]
            rloc = d - lo
            ok = jnp.logical_and(rloc >= 0, rloc < HALF)
            di[pl.ds(v * 16, 16)] = jnp.where(ok, rloc, DUMMY)

    def chunk_step(k, cur, nxt):
        @pl.when(k + 1 < cnt)
        def _():
            wait_idx(nxt)
        mask_dst(cur[0])
        pltpu.sync_copy(onesb, deg_sh.at[cur[0]], add=True)

        @pl.when(k + 2 < cnt)
        def _():
            issue_idx(k + 2, cur)

    issue_idx(0, slots[0])
    wait_idx(slots[0])
    issue_idx(1, slots[1])

    def pair(p, _):
        chunk_step(p * 2, slots[0], slots[1])
        chunk_step(p * 2 + 1, slots[1], slots[0])
        return 0
    lax.fori_loop(0, 195, pair, 0)

    @pl.when(cnt == 391)
    def _():
        chunk_step(390, slots[0], slots[1])

    plsc.subcore_barrier()
    g0 = lo + l0

    @pl.when(s < NS - 1)
    def _():
        pltpu.sync_copy(deg_sh.at[pl.ds(l0, TROWS)], deg_out.at[pl.ds(g0, TROWS)])

    @pl.when(s == NS - 1)
    def _():
        pltpu.sync_copy(deg_sh.at[pl.ds(l0, LTROWS)], deg_out.at[pl.ds(g0, LTROWS)])


@functools.partial(
    pl.kernel,
    out_type=jax.ShapeDtypeStruct((N, D), jnp.float32),
    mesh=_MESH,
    compiler_params=pltpu.CompilerParams(use_tc_tiling_on_sc=False, needs_layout_passes=False),
    scratch_types=[
        pltpu.VMEM((CPE,), jnp.int32),
        pltpu.VMEM((CPE,), jnp.int32),
        pltpu.VMEM((CPE,), jnp.int32),
        pltpu.VMEM((CPE,), jnp.int32),
        pltpu.VMEM((CPE, D), jnp.float32),
        pltpu.VMEM((CPE, D), jnp.float32),
        pltpu.VMEM((112, D), jnp.float32),
        pltpu.VMEM_SHARED((SROWS, D), jnp.float32),
        pltpu.SemaphoreType.DMA,
        pltpu.SemaphoreType.DMA,
        pltpu.SemaphoreType.DMA,
        pltpu.SemaphoreType.DMA,
        pltpu.SemaphoreType.DMA,
        pltpu.SemaphoreType.DMA,
    ],
)
def _agg(y_hbm, src_hbm, dst_hbm, agg_out,
         si0, si1, di0, di1, r0, r1, zbuf,
         agg_sh, ss0, ss1, ds0, ds1, gs0, gs1):
    c = lax.axis_index("c")
    s = lax.axis_index("s")
    lo = c * HALF
    start = s * 390 + jnp.minimum(s, 10)
    cnt = jnp.where(s < 10, 391, 390)
    slots = ((si0, di0, r0, ss0, ds0, gs0), (si1, di1, r1, ss1, ds1, gs1))
    z16 = jnp.zeros((16,), jnp.float32)
    l0 = pl.multiple_of(s * TROWS, 8)

    def _zb(r, _):
        for u in range(D // 16):
            zbuf[r, pl.ds(u * 16, 16)] = z16
        return 0
    lax.fori_loop(0, 112, _zb, 0)

    for q in range(TROWS // 112):
        pltpu.sync_copy(zbuf, agg_sh.at[pl.ds(l0 + q * 112, 112)])
    plsc.subcore_barrier()

    def issue_idx(k, slot):
        base = pl.multiple_of((start + k) * CPE, 8)
        pltpu.async_copy(src_hbm.at[pl.ds(base, CPE)], slot[0], slot[3])
        pltpu.async_copy(dst_hbm.at[pl.ds(base, CPE)], slot[1], slot[4])

    def wait_idx(slot):
        pltpu.make_async_copy(src_hbm.at[pl.ds(0, CPE)], slot[0], slot[3]).wait()
        pltpu.make_async_copy(dst_hbm.at[pl.ds(0, CPE)], slot[1], slot[4]).wait()

    def issue_gather(slot):
        pltpu.async_copy(y_hbm.at[slot[0]], slot[2], slot[5])

    def wait_gather(slot):
        pltpu.make_async_copy(y_hbm.at[slot[0]], slot[2], slot[5]).wait()

    def mask_dst(slot):
        di = slot[1]
        for v in range(CPE // 16):
            d = di[pl.ds(v * 16, 16)]
            rloc = d - lo
            ok = jnp.logical_and(rloc >= 0, rloc < HALF)
            di[pl.ds(v * 16, 16)] = jnp.where(ok, rloc, DUMMY)

    def chunk_step(k, cur, nxt):
        @pl.when(k + 1 < cnt)
        def _():
            wait_idx(nxt)
            issue_gather(nxt)
        wait_gather(cur)
        mask_dst(cur)
        pltpu.sync_copy(cur[2], agg_sh.at[cur[1]], add=True)

        @pl.when(k + 2 < cnt)
        def _():
            issue_idx(k + 2, cur)

    issue_idx(0, slots[0])
    wait_idx(slots[0])
    issue_gather(slots[0])
    issue_idx(1, slots[1])

    def pair(p, _):
        chunk_step(p * 2, slots[0], slots[1])
        chunk_step(p * 2 + 1, slots[1], slots[0])
        return 0
    lax.fori_loop(0, 195, pair, 0)

    @pl.when(cnt == 391)
    def _():
        chunk_step(390, slots[0], slots[1])

    plsc.subcore_barrier()

    g0 = lo + l0

    @pl.when(s < NS - 1)
    def _():
        pltpu.sync_copy(agg_sh.at[pl.ds(l0, TROWS)], agg_out.at[pl.ds(g0, TROWS)])

    @pl.when(s == NS - 1)
    def _():
        pltpu.sync_copy(agg_sh.at[pl.ds(l0, LTROWS)], agg_out.at[pl.ds(g0, LTROWS)])


@functools.partial(
    pl.kernel,
    out_type=jax.ShapeDtypeStruct((E,), jnp.float32),
    mesh=_MESH,
    compiler_params=pltpu.CompilerParams(use_tc_tiling_on_sc=False, needs_layout_passes=False),
    scratch_types=[
        pltpu.VMEM((CPE,), jnp.int32),
        pltpu.VMEM((CPE,), jnp.int32),
        pltpu.VMEM((CPE,), jnp.int32),
        pltpu.VMEM((CPE,), jnp.int32),
        pltpu.VMEM((CPE, D), jnp.float32),
        pltpu.VMEM((CPE, D), jnp.float32),
        pltpu.VMEM((CPE, D), jnp.float32),
        pltpu.VMEM((CPE, D), jnp.float32),
        pltpu.VMEM((196 * CPE,), jnp.float32),
        pltpu.SemaphoreType.DMA,
        pltpu.SemaphoreType.DMA,
        pltpu.SemaphoreType.DMA,
        pltpu.SemaphoreType.DMA,
        pltpu.SemaphoreType.DMA,
        pltpu.SemaphoreType.DMA,
        pltpu.SemaphoreType.DMA,
        pltpu.SemaphoreType.DMA,
    ],
)
def _rate(ref_hbm, src_hbm, dst_hbm, out_hbm,
          si0, si1, di0, di1, ra0, ra1, rb0, rb1, outbuf,
          ss0, ss1, ds0, ds1, ga0, ga1, gb0, gb1):
    c = lax.axis_index("c")
    s = lax.axis_index("s")
    w = c * NS + s
    start = w * 195 + jnp.minimum(w, 10)
    cnt = jnp.where(w < 10, 196, 195)
    slots = (
        (si0, di0, ra0, rb0, ss0, ds0, ga0, gb0),
        (si1, di1, ra1, rb1, ss1, ds1, ga1, gb1),
    )

    def issue_idx(k, slot):
        base = pl.multiple_of((start + k) * CPE, 8)
        pltpu.async_copy(src_hbm.at[pl.ds(base, CPE)], slot[0], slot[4])
        pltpu.async_copy(dst_hbm.at[pl.ds(base, CPE)], slot[1], slot[5])

    def wait_idx(slot):
        pltpu.make_async_copy(src_hbm.at[pl.ds(0, CPE)], slot[0], slot[4]).wait()
        pltpu.make_async_copy(dst_hbm.at[pl.ds(0, CPE)], slot[1], slot[5]).wait()

    def issue_gather(slot):
        pltpu.async_copy(ref_hbm.at[slot[0]], slot[2], slot[6])
        pltpu.async_copy(ref_hbm.at[slot[1]], slot[3], slot[7])

    def wait_gather(slot):
        pltpu.make_async_copy(ref_hbm.at[slot[0]], slot[2], slot[6]).wait()
        pltpu.make_async_copy(ref_hbm.at[slot[1]], slot[3], slot[7]).wait()

    iota16 = lax.iota(jnp.int32, 16)

    def dots(k, slot):
        ra, rb = slot[2], slot[3]
        ob = k * CPE

        def grp(g, _):
            rows = g * 16 + iota16

            def dd(d, acc):
                dv = jnp.broadcast_to(d, (16,))
                a = plsc.load_gather(ra, [rows, dv])
                b = plsc.load_gather(rb, [rows, dv])
                return acc + a * b
            acc = lax.fori_loop(0, D, dd, jnp.zeros((16,), jnp.float32))
            outbuf[pl.ds(ob + g * 16, 16)] = acc
            return 0
        lax.fori_loop(0, CPE // 16, grp, 0)

    def chunk_step(k, cur, nxt):
        @pl.when(k + 1 < cnt)
        def _():
            wait_idx(nxt)
            issue_gather(nxt)
        wait_gather(cur)
        dots(k, cur)

        @pl.when(k + 2 < cnt)
        def _():
            issue_idx(k + 2, cur)

    issue_idx(0, slots[0])
    wait_idx(slots[0])
    issue_gather(slots[0])
    issue_idx(1, slots[1])

    def pair(p, _):
        chunk_step(p * 2, slots[0], slots[1])
        chunk_step(p * 2 + 1, slots[1], slots[0])
        return 0
    lax.fori_loop(0, 97, pair, 0)

    chunk_step(194, slots[0], slots[1])

    @pl.when(cnt == 196)
    def _():
        chunk_step(195, slots[1], slots[0])

    obase = pl.multiple_of(start * CPE, 8)

    @pl.when(w < 10)
    def _():
        pltpu.sync_copy(outbuf.at[pl.ds(0, 196 * CPE)],
                        out_hbm.at[pl.ds(obase, 196 * CPE)])

    @pl.when(w >= 10)
    def _():
        pltpu.sync_copy(outbuf.at[pl.ds(0, 195 * CPE)],
                        out_hbm.at[pl.ds(obase, 195 * CPE)])


def kernel(edge_index, edge_attr, movie_w, user_w,
           W1_self, W1_neigh, b1, W2_self, W2_neigh, b2):
    src = edge_index[0]
    dst = edge_index[1]
    x = jnp.concatenate([movie_w, user_w], axis=0)
    deg = _deg(dst)
    y1, z1 = _tc_proj((x,), W1_neigh, W1_self, b1.reshape(1, D), do_relu=False)
    agg1 = _agg(y1, src, dst)
    y2, z2 = _tc_proj((z1, agg1, deg), W2_neigh, W2_self,
                      b2.reshape(1, D), do_relu=True)
    agg2 = _agg(y2, src, dst)
    refined = _tc_combine(z2, agg2, deg)
    ratings = _rate(refined, src, dst)
    return ratings, refined


# speedup vs baseline: 3.2949x; 1.0028x over previous
import functools

import jax
import jax.numpy as jnp
from jax import lax
from jax.experimental import pallas as pl
from jax.experimental.pallas import tpu as pltpu
from jax.experimental.pallas import tpu_sc as plsc

N = 50000
E = 800000
D = 64
NC, NS = 2, 16
HALF = N // 2
TROWS = 1568
SROWS = NS * TROWS
DUMMY = 25080
CPE = 128
NCHUNK = E // CPE
LTROWS = HALF - (NS - 1) * TROWS

_MESH = plsc.VectorSubcoreMesh(core_axis_name="c", subcore_axis_name="s")


def _tc_proj(operands, w_neigh, w_self, b2d, do_relu):
    R = 2000
    with_agg = len(operands) == 3

    def body(*refs):
        if with_agg:
            z_ref, a_ref, d_ref = refs[0:3]
            wn_ref, ws_ref, b_ref, y_ref, zo_ref = refs[3:]
            inv = 1.0 / jnp.maximum(d_ref[...][:, 0:1], 1.0)
            t = z_ref[...] + a_ref[...] * inv
        else:
            (z_ref, wn_ref, ws_ref, b_ref, y_ref, zo_ref) = refs
            t = z_ref[...]
        if do_relu:
            t = jnp.maximum(t, 0.0)
        y_ref[...] = jnp.dot(t, wn_ref[...], preferred_element_type=jnp.float32)
        zo_ref[...] = (
            jnp.dot(t, ws_ref[...], preferred_element_type=jnp.float32) + b_ref[...]
        )

    row_specs = [pl.BlockSpec((R, D), lambda i: (i, 0))] * (2 if with_agg else 1)
    if with_agg:
        row_specs.append(pl.BlockSpec((R, 16), lambda i: (i, 0)))
    return pl.pallas_call(
        body,
        grid=(N // R,),
        in_specs=row_specs + [
            pl.BlockSpec((D, D), lambda i: (0, 0)),
            pl.BlockSpec((D, D), lambda i: (0, 0)),
            pl.BlockSpec((1, D), lambda i: (0, 0)),
        ],
        out_specs=[
            pl.BlockSpec((R, D), lambda i: (i, 0)),
            pl.BlockSpec((R, D), lambda i: (i, 0)),
        ],
        out_shape=[
            jax.ShapeDtypeStruct((N, D), jnp.float32),
            jax.ShapeDtypeStruct((N, D), jnp.float32),
        ],
    )(*operands, w_neigh, w_self, b2d)


def _tc_combine(z, agg, deg):
    R = 2000

    def body(z_ref, a_ref, d_ref, o_ref):
        inv = 1.0 / jnp.maximum(d_ref[...][:, 0:1], 1.0)
        o_ref[...] = z_ref[...] + a_ref[...] * inv

    return pl.pallas_call(
        body,
        grid=(N // R,),
        in_specs=[
            pl.BlockSpec((R, D), lambda i: (i, 0)),
            pl.BlockSpec((R, D), lambda i: (i, 0)),
            pl.BlockSpec((R, 16), lambda i: (i, 0)),
        ],
        out_specs=pl.BlockSpec((R, D), lambda i: (i, 0)),
        out_shape=jax.ShapeDtypeStruct((N, D), jnp.float32),
    )(z, agg, deg)


@functools.partial(
    pl.kernel,
    out_type=jax.ShapeDtypeStruct((N, 16), jnp.float32),
    mesh=_MESH,
    compiler_params=pltpu.CompilerParams(use_tc_tiling_on_sc=False, needs_layout_passes=False),
    scratch_types=[
        pltpu.VMEM((CPE,), jnp.int32),
        pltpu.VMEM((CPE,), jnp.int32),
        pltpu.VMEM((CPE, 16), jnp.float32),
        pltpu.VMEM((224, 16), jnp.float32),
        pltpu.VMEM_SHARED((SROWS, 16), jnp.float32),
        pltpu.SemaphoreType.DMA,
        pltpu.SemaphoreType.DMA,
    ],
)
def _deg(dst_hbm, deg_out, di0, di1, onesb, zdbuf, deg_sh, ds0, ds1):
    c = lax.axis_index("c")
    s = lax.axis_index("s")
    lo = c * HALF
    start = s * 390 + jnp.minimum(s, 10)
    cnt = jnp.where(s < 10, 391, 390)
    slots = ((di0, ds0), (di1, ds1))
    z16 = jnp.zeros((16,), jnp.float32)
    ones16 = jnp.ones((16,), jnp.float32)
    l0 = pl.multiple_of(s * TROWS, 8)

    def _zb(r, _):
        zdbuf[r, pl.ds(0, 16)] = z16
        return 0
    lax.fori_loop(0, 224, _zb, 0)

    def _ob(r, _):
        onesb[r, pl.ds(0, 16)] = ones16
        return 0
    lax.fori_loop(0, CPE, _ob, 0)

    for q in range(TROWS // 224):
        pltpu.sync_copy(zdbuf, deg_sh.at[pl.ds(l0 + q * 224, 224)])
    plsc.subcore_barrier()

    def issue_idx(k, slot):
        base = pl.multiple_of((start + k) * CPE, 8)
        pltpu.async_copy(dst_hbm.at[pl.ds(base, CPE)], slot[0], slot[1])

    def wait_idx(slot):
        pltpu.make_async_copy(dst_hbm.at[pl.ds(0, CPE)], slot[0], slot[1]).wait()

    def mask_dst(di):
        for v in range(CPE // 16):
            d = di[pl.ds(v * 16, 16)]
            rloc = d - lo
            ok = jnp.logical_and(rloc >= 0, rloc < HALF)
            di[pl.ds(v * 16, 16)] = jnp.where(ok, rloc, DUMMY)

    def chunk_step(k, cur, nxt):
        @pl.when(k + 1 < cnt)
        def _():
            wait_idx(nxt)
        mask_dst(cur[0])
        pltpu.sync_copy(onesb, deg_sh.at[cur[0]], add=True)

        @pl.when(k + 2 < cnt)
        def _():
            issue_idx(k + 2, cur)

    issue_idx(0, slots[0])
    wait_idx(slots[0])
    issue_idx(1, slots[1])

    def pair(p, _):
        chunk_step(p * 2, slots[0], slots[1])
        chunk_step(p * 2 + 1, slots[1], slots[0])
        return 0
    lax.fori_loop(0, 195, pair, 0)

    @pl.when(cnt == 391)
    def _():
        chunk_step(390, slots[0], slots[1])

    plsc.subcore_barrier()
    g0 = lo + l0

    @pl.when(s < NS - 1)
    def _():
        pltpu.sync_copy(deg_sh.at[pl.ds(l0, TROWS)], deg_out.at[pl.ds(g0, TROWS)])

    @pl.when(s == NS - 1)
    def _():
        pltpu.sync_copy(deg_sh.at[pl.ds(l0, LTROWS)], deg_out.at[pl.ds(g0, LTROWS)])


@functools.partial(
    pl.kernel,
    out_type=jax.ShapeDtypeStruct((N, D), jnp.float32),
    mesh=_MESH,
    compiler_params=pltpu.CompilerParams(use_tc_tiling_on_sc=False, needs_layout_passes=False),
    scratch_types=[
        pltpu.VMEM((CPE,), jnp.int32),
        pltpu.VMEM((CPE,), jnp.int32),
        pltpu.VMEM((CPE,), jnp.int32),
        pltpu.VMEM((CPE,), jnp.int32),
        pltpu.VMEM((CPE, D), jnp.float32),
        pltpu.VMEM((CPE, D), jnp.float32),
        pltpu.VMEM((112, D), jnp.float32),
        pltpu.VMEM_SHARED((SROWS, D), jnp.float32),
        pltpu.SemaphoreType.DMA,
        pltpu.SemaphoreType.DMA,
        pltpu.SemaphoreType.DMA,
        pltpu.SemaphoreType.DMA,
        pltpu.SemaphoreType.DMA,
        pltpu.SemaphoreType.DMA,
    ],
)
def _agg(y_hbm, src_hbm, dst_hbm, agg_out,
         si0, si1, di0, di1, r0, r1, zbuf,
         agg_sh, ss0, ss1, ds0, ds1, gs0, gs1):
    c = lax.axis_index("c")
    s = lax.axis_index("s")
    lo = c * HALF
    start = s * 390 + jnp.minimum(s, 10)
    cnt = jnp.where(s < 10, 391, 390)
    slots = ((si0, di0, r0, ss0, ds0, gs0), (si1, di1, r1, ss1, ds1, gs1))
    z16 = jnp.zeros((16,), jnp.float32)
    l0 = pl.multiple_of(s * TROWS, 8)

    def _zb(r, _):
        for u in range(D // 16):
            zbuf[r, pl.ds(u * 16, 16)] = z16
        return 0
    lax.fori_loop(0, 112, _zb, 0)

    for q in range(TROWS // 112):
        pltpu.sync_copy(zbuf, agg_sh.at[pl.ds(l0 + q * 112, 112)])
    plsc.subcore_barrier()

    def issue_idx(k, slot):
        base = pl.multiple_of((start + k) * CPE, 8)
        pltpu.async_copy(src_hbm.at[pl.ds(base, CPE)], slot[0], slot[3])
        pltpu.async_copy(dst_hbm.at[pl.ds(base, CPE)], slot[1], slot[4])

    def wait_idx(slot):
        pltpu.make_async_copy(src_hbm.at[pl.ds(0, CPE)], slot[0], slot[3]).wait()
        pltpu.make_async_copy(dst_hbm.at[pl.ds(0, CPE)], slot[1], slot[4]).wait()

    def issue_gather(slot):
        pltpu.async_copy(y_hbm.at[slot[0]], slot[2], slot[5])

    def wait_gather(slot):
        pltpu.make_async_copy(y_hbm.at[slot[0]], slot[2], slot[5]).wait()

    def mask_dst(slot):
        di = slot[1]
        for v in range(CPE // 16):
            d = di[pl.ds(v * 16, 16)]
            rloc = d - lo
            ok = jnp.logical_and(rloc >= 0, rloc < HALF)
            di[pl.ds(v * 16, 16)] = jnp.where(ok, rloc, DUMMY)

    def chunk_step(k, cur, nxt):
        @pl.when(k + 1 < cnt)
        def _():
            wait_idx(nxt)
            issue_gather(nxt)
        wait_gather(cur)
        mask_dst(cur)
        pltpu.sync_copy(cur[2], agg_sh.at[cur[1]], add=True)

        @pl.when(k + 2 < cnt)
        def _():
            issue_idx(k + 2, cur)

    issue_idx(0, slots[0])
    wait_idx(slots[0])
    issue_gather(slots[0])
    issue_idx(1, slots[1])

    def pair(p, _):
        chunk_step(p * 2, slots[0], slots[1])
        chunk_step(p * 2 + 1, slots[1], slots[0])
        return 0
    lax.fori_loop(0, 195, pair, 0)

    @pl.when(cnt == 391)
    def _():
        chunk_step(390, slots[0], slots[1])

    plsc.subcore_barrier()

    g0 = lo + l0

    @pl.when(s < NS - 1)
    def _():
        pltpu.sync_copy(agg_sh.at[pl.ds(l0, TROWS)], agg_out.at[pl.ds(g0, TROWS)])

    @pl.when(s == NS - 1)
    def _():
        pltpu.sync_copy(agg_sh.at[pl.ds(l0, LTROWS)], agg_out.at[pl.ds(g0, LTROWS)])


@functools.partial(
    pl.kernel,
    out_type=jax.ShapeDtypeStruct((E,), jnp.float32),
    mesh=_MESH,
    compiler_params=pltpu.CompilerParams(use_tc_tiling_on_sc=False, needs_layout_passes=False),
    scratch_types=[
        pltpu.VMEM((CPE,), jnp.int32),
        pltpu.VMEM((CPE,), jnp.int32),
        pltpu.VMEM((CPE,), jnp.int32),
        pltpu.VMEM((CPE,), jnp.int32),
        pltpu.VMEM((CPE, D), jnp.float32),
        pltpu.VMEM((CPE, D), jnp.float32),
        pltpu.VMEM((CPE, D), jnp.float32),
        pltpu.VMEM((CPE, D), jnp.float32),
        pltpu.VMEM((196 * CPE,), jnp.float32),
        pltpu.SemaphoreType.DMA,
        pltpu.SemaphoreType.DMA,
        pltpu.SemaphoreType.DMA,
        pltpu.SemaphoreType.DMA,
        pltpu.SemaphoreType.DMA,
        pltpu.SemaphoreType.DMA,
        pltpu.SemaphoreType.DMA,
        pltpu.SemaphoreType.DMA,
    ],
)
def _rate(ref_hbm, src_hbm, dst_hbm, out_hbm,
          si0, si1, di0, di1, ra0, ra1, rb0, rb1, outbuf,
          ss0, ss1, ds0, ds1, ga0, ga1, gb0, gb1):
    c = lax.axis_index("c")
    s = lax.axis_index("s")
    w = c * NS + s
    start = w * 195 + jnp.minimum(w, 10)
    cnt = jnp.where(w < 10, 196, 195)
    slots = (
        (si0, di0, ra0, rb0, ss0, ds0, ga0, gb0),
        (si1, di1, ra1, rb1, ss1, ds1, ga1, gb1),
    )

    def issue_idx(k, slot):
        base = pl.multiple_of((start + k) * CPE, 8)
        pltpu.async_copy(src_hbm.at[pl.ds(base, CPE)], slot[0], slot[4])
        pltpu.async_copy(dst_hbm.at[pl.ds(base, CPE)], slot[1], slot[5])

    def wait_idx(slot):
        pltpu.make_async_copy(src_hbm.at[pl.ds(0, CPE)], slot[0], slot[4]).wait()
        pltpu.make_async_copy(dst_hbm.at[pl.ds(0, CPE)], slot[1], slot[5]).wait()

    def issue_gather(slot):
        pltpu.async_copy(ref_hbm.at[slot[0]], slot[2], slot[6])
        pltpu.async_copy(ref_hbm.at[slot[1]], slot[3], slot[7])

    def wait_gather(slot):
        pltpu.make_async_copy(ref_hbm.at[slot[0]], slot[2], slot[6]).wait()
        pltpu.make_async_copy(ref_hbm.at[slot[1]], slot[3], slot[7]).wait()

    iota16 = lax.iota(jnp.int32, 16)

    def dots(k, slot):
        ra, rb = slot[2], slot[3]
        ob = k * CPE

        def grp(g, _):
            rows = g * 16 + iota16
            acc = jnp.zeros((16,), jnp.float32)
            for d in range(D):
                dv = jnp.full((16,), d, jnp.int32)
                a = plsc.load_gather(ra, [rows, dv])
                b = plsc.load_gather(rb, [rows, dv])
                acc = acc + a * b
            outbuf[pl.ds(ob + g * 16, 16)] = acc
            return 0
        lax.fori_loop(0, CPE // 16, grp, 0)

    def chunk_step(k, cur, nxt):
        @pl.when(k + 1 < cnt)
        def _():
            wait_idx(nxt)
            issue_gather(nxt)
        wait_gather(cur)
        dots(k, cur)

        @pl.when(k + 2 < cnt)
        def _():
            issue_idx(k + 2, cur)

    issue_idx(0, slots[0])
    wait_idx(slots[0])
    issue_gather(slots[0])
    issue_idx(1, slots[1])

    def pair(p, _):
        chunk_step(p * 2, slots[0], slots[1])
        chunk_step(p * 2 + 1, slots[1], slots[0])
        return 0
    lax.fori_loop(0, 97, pair, 0)

    chunk_step(194, slots[0], slots[1])

    @pl.when(cnt == 196)
    def _():
        chunk_step(195, slots[1], slots[0])

    obase = pl.multiple_of(start * CPE, 8)

    @pl.when(w < 10)
    def _():
        pltpu.sync_copy(outbuf.at[pl.ds(0, 196 * CPE)],
                        out_hbm.at[pl.ds(obase, 196 * CPE)])

    @pl.when(w >= 10)
    def _():
        pltpu.sync_copy(outbuf.at[pl.ds(0, 195 * CPE)],
                        out_hbm.at[pl.ds(obase, 195 * CPE)])


def kernel(edge_index, edge_attr, movie_w, user_w,
           W1_self, W1_neigh, b1, W2_self, W2_neigh, b2):
    src = edge_index[0]
    dst = edge_index[1]
    x = jnp.concatenate([movie_w, user_w], axis=0)
    deg = _deg(dst)
    y1, z1 = _tc_proj((x,), W1_neigh, W1_self, b1.reshape(1, D), do_relu=False)
    agg1 = _agg(y1, src, dst)
    y2, z2 = _tc_proj((z1, agg1, deg), W2_neigh, W2_self,
                      b2.reshape(1, D), do_relu=True)
    agg2 = _agg(y2, src, dst)
    refined = _tc_combine(z2, agg2, deg)
    ratings = _rate(refined, src, dst)
    return ratings, refined


# speedup vs baseline: 5.5819x; 1.6941x over previous
import functools

import jax
import jax.numpy as jnp
from jax import lax
from jax.experimental import pallas as pl
from jax.experimental.pallas import tpu as pltpu
from jax.experimental.pallas import tpu_sc as plsc

N = 50000
E = 800000
D = 64
NC, NS = 2, 16
HALF = N // 2
TROWS = 1568
SROWS = NS * TROWS
DUMMY = 25080
CPE = 128
NCHUNK = E // CPE
LTROWS = HALF - (NS - 1) * TROWS

_MESH = plsc.VectorSubcoreMesh(core_axis_name="c", subcore_axis_name="s")


def _tc_proj(operands, w_neigh, w_self, b2d, do_relu):
    R = 2000
    with_agg = len(operands) == 3

    def body(*refs):
        if with_agg:
            z_ref, a_ref, d_ref = refs[0:3]
            wn_ref, ws_ref, b_ref, y_ref, zo_ref = refs[3:]
            inv = 1.0 / jnp.maximum(d_ref[...][:, 0:1], 1.0)
            t = z_ref[...] + a_ref[...] * inv
        else:
            (z_ref, wn_ref, ws_ref, b_ref, y_ref, zo_ref) = refs
            t = z_ref[...]
        if do_relu:
            t = jnp.maximum(t, 0.0)
        y_ref[...] = jnp.dot(t, wn_ref[...], preferred_element_type=jnp.float32)
        zo_ref[...] = (
            jnp.dot(t, ws_ref[...], preferred_element_type=jnp.float32) + b_ref[...]
        )

    row_specs = [pl.BlockSpec((R, D), lambda i: (i, 0))] * (2 if with_agg else 1)
    if with_agg:
        row_specs.append(pl.BlockSpec((R, 16), lambda i: (i, 0)))
    return pl.pallas_call(
        body,
        grid=(N // R,),
        in_specs=row_specs + [
            pl.BlockSpec((D, D), lambda i: (0, 0)),
            pl.BlockSpec((D, D), lambda i: (0, 0)),
            pl.BlockSpec((1, D), lambda i: (0, 0)),
        ],
        out_specs=[
            pl.BlockSpec((R, D), lambda i: (i, 0)),
            pl.BlockSpec((R, D), lambda i: (i, 0)),
        ],
        out_shape=[
            jax.ShapeDtypeStruct((N, D), jnp.float32),
            jax.ShapeDtypeStruct((N, D), jnp.float32),
        ],
    )(*operands, w_neigh, w_self, b2d)


def _tc_combine(z, agg, deg):
    R = 2000

    def body(z_ref, a_ref, d_ref, o_ref):
        inv = 1.0 / jnp.maximum(d_ref[...][:, 0:1], 1.0)
        o_ref[...] = z_ref[...] + a_ref[...] * inv

    return pl.pallas_call(
        body,
        grid=(N // R,),
        in_specs=[
            pl.BlockSpec((R, D), lambda i: (i, 0)),
            pl.BlockSpec((R, D), lambda i: (i, 0)),
            pl.BlockSpec((R, 16), lambda i: (i, 0)),
        ],
        out_specs=pl.BlockSpec((R, D), lambda i: (i, 0)),
        out_shape=jax.ShapeDtypeStruct((N, D), jnp.float32),
    )(z, agg, deg)


@functools.partial(
    pl.kernel,
    out_type=jax.ShapeDtypeStruct((N, 16), jnp.float32),
    mesh=_MESH,
    compiler_params=pltpu.CompilerParams(use_tc_tiling_on_sc=False, needs_layout_passes=False),
    scratch_types=[
        pltpu.VMEM((CPE,), jnp.int32),
        pltpu.VMEM((CPE,), jnp.int32),
        pltpu.VMEM((CPE, 16), jnp.float32),
        pltpu.VMEM((224, 16), jnp.float32),
        pltpu.VMEM_SHARED((SROWS, 16), jnp.float32),
        pltpu.SemaphoreType.DMA,
        pltpu.SemaphoreType.DMA,
    ],
)
def _deg(dst_hbm, deg_out, di0, di1, onesb, zdbuf, deg_sh, ds0, ds1):
    c = lax.axis_index("c")
    s = lax.axis_index("s")
    lo = c * HALF
    start = s * 390 + jnp.minimum(s, 10)
    cnt = jnp.where(s < 10, 391, 390)
    slots = ((di0, ds0), (di1, ds1))
    z16 = jnp.zeros((16,), jnp.float32)
    ones16 = jnp.ones((16,), jnp.float32)
    l0 = pl.multiple_of(s * TROWS, 8)

    def _zb(r, _):
        zdbuf[r, pl.ds(0, 16)] = z16
        return 0
    lax.fori_loop(0, 224, _zb, 0)

    def _ob(r, _):
        onesb[r, pl.ds(0, 16)] = ones16
        return 0
    lax.fori_loop(0, CPE, _ob, 0)

    for q in range(TROWS // 224):
        pltpu.sync_copy(zdbuf, deg_sh.at[pl.ds(l0 + q * 224, 224)])
    plsc.subcore_barrier()

    def issue_idx(k, slot):
        base = pl.multiple_of((start + k) * CPE, 8)
        pltpu.async_copy(dst_hbm.at[pl.ds(base, CPE)], slot[0], slot[1])

    def wait_idx(slot):
        pltpu.make_async_copy(dst_hbm.at[pl.ds(0, CPE)], slot[0], slot[1]).wait()

    def mask_dst(di):
        for v in range(CPE // 16):
            d = di[pl.ds(v * 16, 16)]
            rloc = d - lo
            ok = jnp.logical_and(rloc >= 0, rloc < HALF)
            di[pl.ds(v * 16, 16)] = jnp.where(ok, rloc, DUMMY)

    def chunk_step(k, cur, nxt):
        @pl.when(k + 1 < cnt)
        def _():
            wait_idx(nxt)
        mask_dst(cur[0])
        pltpu.sync_copy(onesb, deg_sh.at[cur[0]], add=True)

        @pl.when(k + 2 < cnt)
        def _():
            issue_idx(k + 2, cur)

    issue_idx(0, slots[0])
    wait_idx(slots[0])
    issue_idx(1, slots[1])

    def pair(p, _):
        chunk_step(p * 2, slots[0], slots[1])
        chunk_step(p * 2 + 1, slots[1], slots[0])
        return 0
    lax.fori_loop(0, 195, pair, 0)

    @pl.when(cnt == 391)
    def _():
        chunk_step(390, slots[0], slots[1])

    plsc.subcore_barrier()
    g0 = lo + l0

    @pl.when(s < NS - 1)
    def _():
        pltpu.sync_copy(deg_sh.at[pl.ds(l0, TROWS)], deg_out.at[pl.ds(g0, TROWS)])

    @pl.when(s == NS - 1)
    def _():
        pltpu.sync_copy(deg_sh.at[pl.ds(l0, LTROWS)], deg_out.at[pl.ds(g0, LTROWS)])


@functools.partial(
    pl.kernel,
    out_type=jax.ShapeDtypeStruct((N, D), jnp.float32),
    mesh=_MESH,
    compiler_params=pltpu.CompilerParams(use_tc_tiling_on_sc=False, needs_layout_passes=False),
    scratch_types=[
        pltpu.VMEM((CPE,), jnp.int32),
        pltpu.VMEM((CPE,), jnp.int32),
        pltpu.VMEM((CPE,), jnp.int32),
        pltpu.VMEM((CPE,), jnp.int32),
        pltpu.VMEM((CPE, D), jnp.float32),
        pltpu.VMEM((CPE, D), jnp.float32),
        pltpu.VMEM((112, D), jnp.float32),
        pltpu.VMEM_SHARED((SROWS, D), jnp.float32),
        pltpu.SemaphoreType.DMA,
        pltpu.SemaphoreType.DMA,
        pltpu.SemaphoreType.DMA,
        pltpu.SemaphoreType.DMA,
        pltpu.SemaphoreType.DMA,
        pltpu.SemaphoreType.DMA,
    ],
)
def _agg(y_hbm, src_hbm, dst_hbm, agg_out,
         si0, si1, di0, di1, r0, r1, zbuf,
         agg_sh, ss0, ss1, ds0, ds1, gs0, gs1):
    c = lax.axis_index("c")
    s = lax.axis_index("s")
    lo = c * HALF
    start = s * 390 + jnp.minimum(s, 10)
    cnt = jnp.where(s < 10, 391, 390)
    slots = ((si0, di0, r0, ss0, ds0, gs0), (si1, di1, r1, ss1, ds1, gs1))
    z16 = jnp.zeros((16,), jnp.float32)
    l0 = pl.multiple_of(s * TROWS, 8)

    def _zb(r, _):
        for u in range(D // 16):
            zbuf[r, pl.ds(u * 16, 16)] = z16
        return 0
    lax.fori_loop(0, 112, _zb, 0)

    for q in range(TROWS // 112):
        pltpu.sync_copy(zbuf, agg_sh.at[pl.ds(l0 + q * 112, 112)])
    plsc.subcore_barrier()

    def issue_idx(k, slot):
        base = pl.multiple_of((start + k) * CPE, 8)
        pltpu.async_copy(src_hbm.at[pl.ds(base, CPE)], slot[0], slot[3])
        pltpu.async_copy(dst_hbm.at[pl.ds(base, CPE)], slot[1], slot[4])

    def wait_idx(slot):
        pltpu.make_async_copy(src_hbm.at[pl.ds(0, CPE)], slot[0], slot[3]).wait()
        pltpu.make_async_copy(dst_hbm.at[pl.ds(0, CPE)], slot[1], slot[4]).wait()

    def issue_gather(slot):
        pltpu.async_copy(y_hbm.at[slot[0]], slot[2], slot[5])

    def wait_gather(slot):
        pltpu.make_async_copy(y_hbm.at[slot[0]], slot[2], slot[5]).wait()

    def mask_dst(slot):
        di = slot[1]
        for v in range(CPE // 16):
            d = di[pl.ds(v * 16, 16)]
            rloc = d - lo
            ok = jnp.logical_and(rloc >= 0, rloc < HALF)
            di[pl.ds(v * 16, 16)] = jnp.where(ok, rloc, DUMMY)

    def chunk_step(k, cur, nxt):
        @pl.when(k + 1 < cnt)
        def _():
            wait_idx(nxt)
            issue_gather(nxt)
        wait_gather(cur)
        mask_dst(cur)
        pltpu.sync_copy(cur[2], agg_sh.at[cur[1]], add=True)

        @pl.when(k + 2 < cnt)
        def _():
            issue_idx(k + 2, cur)

    issue_idx(0, slots[0])
    wait_idx(slots[0])
    issue_gather(slots[0])
    issue_idx(1, slots[1])

    def pair(p, _):
        chunk_step(p * 2, slots[0], slots[1])
        chunk_step(p * 2 + 1, slots[1], slots[0])
        return 0
    lax.fori_loop(0, 195, pair, 0)

    @pl.when(cnt == 391)
    def _():
        chunk_step(390, slots[0], slots[1])

    plsc.subcore_barrier()

    g0 = lo + l0

    @pl.when(s < NS - 1)
    def _():
        pltpu.sync_copy(agg_sh.at[pl.ds(l0, TROWS)], agg_out.at[pl.ds(g0, TROWS)])

    @pl.when(s == NS - 1)
    def _():
        pltpu.sync_copy(agg_sh.at[pl.ds(l0, LTROWS)], agg_out.at[pl.ds(g0, LTROWS)])


@functools.partial(
    pl.kernel,
    out_type=jax.ShapeDtypeStruct((E,), jnp.float32),
    mesh=_MESH,
    compiler_params=pltpu.CompilerParams(use_tc_tiling_on_sc=False, needs_layout_passes=False),
    scratch_types=[
        pltpu.VMEM((CPE,), jnp.int32),
        pltpu.VMEM((CPE,), jnp.int32),
        pltpu.VMEM((CPE,), jnp.int32),
        pltpu.VMEM((CPE,), jnp.int32),
        pltpu.VMEM((CPE, D), jnp.float32),
        pltpu.VMEM((CPE, D), jnp.float32),
        pltpu.VMEM((CPE, D), jnp.float32),
        pltpu.VMEM((CPE, D), jnp.float32),
        pltpu.VMEM((196 * CPE,), jnp.float32),
        pltpu.SemaphoreType.DMA,
        pltpu.SemaphoreType.DMA,
        pltpu.SemaphoreType.DMA,
        pltpu.SemaphoreType.DMA,
        pltpu.SemaphoreType.DMA,
        pltpu.SemaphoreType.DMA,
        pltpu.SemaphoreType.DMA,
        pltpu.SemaphoreType.DMA,
    ],
)
def _rate(ref_hbm, src_hbm, dst_hbm, out_hbm,
          si0, si1, di0, di1, ra0, ra1, rb0, rb1, outbuf,
          ss0, ss1, ds0, ds1, ga0, ga1, gb0, gb1):
    c = lax.axis_index("c")
    s = lax.axis_index("s")
    w = c * NS + s
    start = w * 195 + jnp.minimum(w, 10)
    cnt = jnp.where(w < 10, 196, 195)
    slots = (
        (si0, di0, ra0, rb0, ss0, ds0, ga0, gb0),
        (si1, di1, ra1, rb1, ss1, ds1, ga1, gb1),
    )

    def issue_idx(k, slot):
        base = pl.multiple_of((start + k) * CPE, 8)
        pltpu.async_copy(src_hbm.at[pl.ds(base, CPE)], slot[0], slot[4])
        pltpu.async_copy(dst_hbm.at[pl.ds(base, CPE)], slot[1], slot[5])

    def wait_idx(slot):
        pltpu.make_async_copy(src_hbm.at[pl.ds(0, CPE)], slot[0], slot[4]).wait()
        pltpu.make_async_copy(dst_hbm.at[pl.ds(0, CPE)], slot[1], slot[5]).wait()

    def issue_gather(slot):
        pltpu.async_copy(ref_hbm.at[slot[0]], slot[2], slot[6])
        pltpu.async_copy(ref_hbm.at[slot[1]], slot[3], slot[7])

    def wait_gather(slot):
        pltpu.make_async_copy(ref_hbm.at[slot[0]], slot[2], slot[6]).wait()
        pltpu.make_async_copy(ref_hbm.at[slot[1]], slot[3], slot[7]).wait()

    iota16 = lax.iota(jnp.int32, 16)

    def dots(k, slot):
        ra, rb = slot[2], slot[3]
        ob = k * CPE

        def grp(g, _):
            res = jnp.zeros((16,), jnp.float32)
            for e in range(16):
                row = g * 16 + e
                p = ra[row, pl.ds(0, 16)] * rb[row, pl.ds(0, 16)]
                for u in range(1, D // 16):
                    sl = pl.ds(u * 16, 16)
                    p = p + ra[row, sl] * rb[row, sl]
                res = jnp.where(iota16 == e, jnp.sum(p), res)
            outbuf[pl.ds(ob + g * 16, 16)] = res
            return 0
        lax.fori_loop(0, CPE // 16, grp, 0)

    def chunk_step(k, cur, nxt):
        @pl.when(k + 1 < cnt)
        def _():
            wait_idx(nxt)
            issue_gather(nxt)
        wait_gather(cur)
        dots(k, cur)

        @pl.when(k + 2 < cnt)
        def _():
            issue_idx(k + 2, cur)

    issue_idx(0, slots[0])
    wait_idx(slots[0])
    issue_gather(slots[0])
    issue_idx(1, slots[1])

    def pair(p, _):
        chunk_step(p * 2, slots[0], slots[1])
        chunk_step(p * 2 + 1, slots[1], slots[0])
        return 0
    lax.fori_loop(0, 97, pair, 0)

    chunk_step(194, slots[0], slots[1])

    @pl.when(cnt == 196)
    def _():
        chunk_step(195, slots[1], slots[0])

    obase = pl.multiple_of(start * CPE, 8)

    @pl.when(w < 10)
    def _():
        pltpu.sync_copy(outbuf.at[pl.ds(0, 196 * CPE)],
                        out_hbm.at[pl.ds(obase, 196 * CPE)])

    @pl.when(w >= 10)
    def _():
        pltpu.sync_copy(outbuf.at[pl.ds(0, 195 * CPE)],
                        out_hbm.at[pl.ds(obase, 195 * CPE)])


def kernel(edge_index, edge_attr, movie_w, user_w,
           W1_self, W1_neigh, b1, W2_self, W2_neigh, b2):
    src = edge_index[0]
    dst = edge_index[1]
    x = jnp.concatenate([movie_w, user_w], axis=0)
    deg = _deg(dst)
    y1, z1 = _tc_proj((x,), W1_neigh, W1_self, b1.reshape(1, D), do_relu=False)
    agg1 = _agg(y1, src, dst)
    y2, z2 = _tc_proj((z1, agg1, deg), W2_neigh, W2_self,
                      b2.reshape(1, D), do_relu=True)
    agg2 = _agg(y2, src, dst)
    refined = _tc_combine(z2, agg2, deg)
    ratings = _rate(refined, src, dst)
    return ratings, refined


# speedup vs baseline: 5.6057x; 1.0043x over previous
import functools

import jax
import jax.numpy as jnp
from jax import lax
from jax.experimental import pallas as pl
from jax.experimental.pallas import tpu as pltpu
from jax.experimental.pallas import tpu_sc as plsc

N = 50000
E = 800000
D = 64
NC, NS = 2, 16
HALF = N // 2
TROWS = 1568
SROWS = NS * TROWS
DUMMY = 25080
CPE = 128
NCHUNK = E // CPE
LTROWS = HALF - (NS - 1) * TROWS

_MESH = plsc.VectorSubcoreMesh(core_axis_name="c", subcore_axis_name="s")


def _tc_proj(operands, w_neigh, w_self, b2d, do_relu):
    R = 2000
    with_agg = len(operands) == 3

    def body(*refs):
        if with_agg:
            z_ref, a_ref, d_ref = refs[0:3]
            wn_ref, ws_ref, b_ref, y_ref, zo_ref = refs[3:]
            inv = 1.0 / jnp.maximum(d_ref[...][:, 0:1], 1.0)
            t = z_ref[...] + a_ref[...] * inv
        else:
            (z_ref, wn_ref, ws_ref, b_ref, y_ref, zo_ref) = refs
            t = z_ref[...]
        if do_relu:
            t = jnp.maximum(t, 0.0)
        y_ref[...] = jnp.dot(t, wn_ref[...], preferred_element_type=jnp.float32)
        zo_ref[...] = (
            jnp.dot(t, ws_ref[...], preferred_element_type=jnp.float32) + b_ref[...]
        )

    row_specs = [pl.BlockSpec((R, D), lambda i: (i, 0))] * (2 if with_agg else 1)
    if with_agg:
        row_specs.append(pl.BlockSpec((R, 16), lambda i: (i, 0)))
    return pl.pallas_call(
        body,
        grid=(N // R,),
        in_specs=row_specs + [
            pl.BlockSpec((D, D), lambda i: (0, 0)),
            pl.BlockSpec((D, D), lambda i: (0, 0)),
            pl.BlockSpec((1, D), lambda i: (0, 0)),
        ],
        out_specs=[
            pl.BlockSpec((R, D), lambda i: (i, 0)),
            pl.BlockSpec((R, D), lambda i: (i, 0)),
        ],
        out_shape=[
            jax.ShapeDtypeStruct((N, D), jnp.float32),
            jax.ShapeDtypeStruct((N, D), jnp.float32),
        ],
    )(*operands, w_neigh, w_self, b2d)


def _tc_combine(z, agg, deg):
    R = 2000

    def body(z_ref, a_ref, d_ref, o_ref):
        inv = 1.0 / jnp.maximum(d_ref[...][:, 0:1], 1.0)
        o_ref[...] = z_ref[...] + a_ref[...] * inv

    return pl.pallas_call(
        body,
        grid=(N // R,),
        in_specs=[
            pl.BlockSpec((R, D), lambda i: (i, 0)),
            pl.BlockSpec((R, D), lambda i: (i, 0)),
            pl.BlockSpec((R, 16), lambda i: (i, 0)),
        ],
        out_specs=pl.BlockSpec((R, D), lambda i: (i, 0)),
        out_shape=jax.ShapeDtypeStruct((N, D), jnp.float32),
    )(z, agg, deg)


@functools.partial(
    pl.kernel,
    out_type=jax.ShapeDtypeStruct((N, 16), jnp.float32),
    mesh=_MESH,
    compiler_params=pltpu.CompilerParams(use_tc_tiling_on_sc=False, needs_layout_passes=False),
    scratch_types=[
        pltpu.VMEM((CPE,), jnp.int32),
        pltpu.VMEM((CPE,), jnp.int32),
        pltpu.VMEM((CPE,), jnp.int32),
        pltpu.VMEM((CPE,), jnp.int32),
        pltpu.VMEM((CPE,), jnp.int32),
        pltpu.VMEM((CPE,), jnp.int32),
        pltpu.VMEM((CPE, 16), jnp.float32),
        pltpu.VMEM((224, 16), jnp.float32),
        pltpu.VMEM_SHARED((SROWS, 16), jnp.float32),
        pltpu.SemaphoreType.DMA,
        pltpu.SemaphoreType.DMA,
        pltpu.SemaphoreType.DMA,
        pltpu.SemaphoreType.DMA,
        pltpu.SemaphoreType.DMA,
        pltpu.SemaphoreType.DMA,
    ],
)
def _deg(dst_hbm, deg_out, di0, di1, di2, dm0, dm1, dm2, onesb, zdbuf, deg_sh,
         ds0, ds1, ds2, sc0, sc1, sc2):
    c = lax.axis_index("c")
    s = lax.axis_index("s")
    lo = c * HALF
    start = s * 390 + jnp.minimum(s, 10)
    cnt = jnp.where(s < 10, 391, 390)
    slots = ((di0, dm0, ds0, sc0), (di1, dm1, ds1, sc1), (di2, dm2, ds2, sc2))
    z16 = jnp.zeros((16,), jnp.float32)
    ones16 = jnp.ones((16,), jnp.float32)
    l0 = pl.multiple_of(s * TROWS, 8)

    def _zb(r, _):
        zdbuf[r, pl.ds(0, 16)] = z16
        return 0
    lax.fori_loop(0, 224, _zb, 0)

    def _ob(r, _):
        onesb[r, pl.ds(0, 16)] = ones16
        return 0
    lax.fori_loop(0, CPE, _ob, 0)

    for q in range(TROWS // 224):
        pltpu.sync_copy(zdbuf, deg_sh.at[pl.ds(l0 + q * 224, 224)])
    plsc.subcore_barrier()

    def issue_idx(k, slot):
        base = pl.multiple_of((start + k) * CPE, 8)
        pltpu.async_copy(dst_hbm.at[pl.ds(base, CPE)], slot[0], slot[2])

    def wait_idx(slot):
        pltpu.make_async_copy(dst_hbm.at[pl.ds(0, CPE)], slot[0], slot[2]).wait()

    def issue_scatter(slot):
        pltpu.async_copy(onesb, deg_sh.at[slot[1]], slot[3], add=True)

    def wait_scatter(slot):
        pltpu.make_async_copy(onesb, deg_sh.at[slot[1]], slot[3]).wait()

    def mask_dst(slot):
        di, dm = slot[0], slot[1]
        for v in range(CPE // 16):
            d = di[pl.ds(v * 16, 16)]
            rloc = d - lo
            ok = jnp.logical_and(rloc >= 0, rloc < HALF)
            dm[pl.ds(v * 16, 16)] = jnp.where(ok, rloc, DUMMY)

    def chunk_step(k, i0, i1, i2):
        @pl.when(k >= 2)
        def _():
            wait_scatter(i1)
        mask_dst(i0)
        issue_scatter(i0)

        @pl.when(k + 1 < cnt)
        def _():
            wait_idx(i1)

        @pl.when(k + 2 < cnt)
        def _():
            issue_idx(k + 2, i2)

    issue_idx(0, slots[0])
    wait_idx(slots[0])
    issue_idx(1, slots[1])

    def triple(p, _):
        chunk_step(p * 3, slots[0], slots[1], slots[2])
        chunk_step(p * 3 + 1, slots[1], slots[2], slots[0])
        chunk_step(p * 3 + 2, slots[2], slots[0], slots[1])
        return 0
    lax.fori_loop(0, 130, triple, 0)

    @pl.when(cnt == 391)
    def _():
        chunk_step(390, slots[0], slots[1], slots[2])
        wait_scatter(slots[2])
        wait_scatter(slots[0])

    @pl.when(cnt == 390)
    def _():
        wait_scatter(slots[1])
        wait_scatter(slots[2])

    plsc.subcore_barrier()
    g0 = lo + l0

    @pl.when(s < NS - 1)
    def _():
        pltpu.sync_copy(deg_sh.at[pl.ds(l0, TROWS)], deg_out.at[pl.ds(g0, TROWS)])

    @pl.when(s == NS - 1)
    def _():
        pltpu.sync_copy(deg_sh.at[pl.ds(l0, LTROWS)], deg_out.at[pl.ds(g0, LTROWS)])


@functools.partial(
    pl.kernel,
    out_type=jax.ShapeDtypeStruct((N, D), jnp.float32),
    mesh=_MESH,
    compiler_params=pltpu.CompilerParams(use_tc_tiling_on_sc=False, needs_layout_passes=False),
    scratch_types=[
        pltpu.VMEM((CPE,), jnp.int32),
        pltpu.VMEM((CPE,), jnp.int32),
        pltpu.VMEM((CPE,), jnp.int32),
        pltpu.VMEM((CPE,), jnp.int32),
        pltpu.VMEM((CPE,), jnp.int32),
        pltpu.VMEM((CPE,), jnp.int32),
        pltpu.VMEM((CPE,), jnp.int32),
        pltpu.VMEM((CPE,), jnp.int32),
        pltpu.VMEM((CPE,), jnp.int32),
        pltpu.VMEM((CPE, D), jnp.float32),
        pltpu.VMEM((CPE, D), jnp.float32),
        pltpu.VMEM((CPE, D), jnp.float32),
        pltpu.VMEM((56, D), jnp.float32),
        pltpu.VMEM_SHARED((SROWS, D), jnp.float32),
        pltpu.SemaphoreType.DMA,
        pltpu.SemaphoreType.DMA,
        pltpu.SemaphoreType.DMA,
        pltpu.SemaphoreType.DMA,
        pltpu.SemaphoreType.DMA,
        pltpu.SemaphoreType.DMA,
        pltpu.SemaphoreType.DMA,
        pltpu.SemaphoreType.DMA,
        pltpu.SemaphoreType.DMA,
    ],
)
def _agg(y_hbm, src_hbm, dst_hbm, agg_out,
         si0, si1, si2, di0, di1, di2, dm0, dm1, dm2, r0, r1, r2, zbuf,
         agg_sh, is0, is1, is2, gs0, gs1, gs2, sc0, sc1, sc2):
    c = lax.axis_index("c")
    s = lax.axis_index("s")
    lo = c * HALF
    start = s * 390 + jnp.minimum(s, 10)
    cnt = jnp.where(s < 10, 391, 390)
    slots = (
        (si0, di0, dm0, r0, is0, gs0, sc0),
        (si1, di1, dm1, r1, is1, gs1, sc1),
        (si2, di2, dm2, r2, is2, gs2, sc2),
    )
    z16 = jnp.zeros((16,), jnp.float32)
    l0 = pl.multiple_of(s * TROWS, 8)

    def _zb(r, _):
        for u in range(D // 16):
            zbuf[r, pl.ds(u * 16, 16)] = z16
        return 0
    lax.fori_loop(0, 56, _zb, 0)

    for q in range(TROWS // 56):
        pltpu.sync_copy(zbuf, agg_sh.at[pl.ds(l0 + q * 56, 56)])
    plsc.subcore_barrier()

    def issue_idx(k, slot):
        base = pl.multiple_of((start + k) * CPE, 8)
        pltpu.async_copy(src_hbm.at[pl.ds(base, CPE)], slot[0], slot[4])
        pltpu.async_copy(dst_hbm.at[pl.ds(base, CPE)], slot[1], slot[4])

    def wait_idx(slot):
        pltpu.make_async_copy(src_hbm.at[pl.ds(0, CPE)], slot[0], slot[4]).wait()
        pltpu.make_async_copy(dst_hbm.at[pl.ds(0, CPE)], slot[1], slot[4]).wait()

    def issue_gather(slot):
        pltpu.async_copy(y_hbm.at[slot[0]], slot[3], slot[5])

    def wait_gather(slot):
        pltpu.make_async_copy(y_hbm.at[slot[0]], slot[3], slot[5]).wait()

    def issue_scatter(slot):
        pltpu.async_copy(slot[3], agg_sh.at[slot[2]], slot[6], add=True)

    def wait_scatter(slot):
        pltpu.make_async_copy(slot[3], agg_sh.at[slot[2]], slot[6]).wait()

    def mask_dst(slot):
        di, dm = slot[1], slot[2]
        for v in range(CPE // 16):
            d = di[pl.ds(v * 16, 16)]
            rloc = d - lo
            ok = jnp.logical_and(rloc >= 0, rloc < HALF)
            dm[pl.ds(v * 16, 16)] = jnp.where(ok, rloc, DUMMY)

    def chunk_step(k, i0, i1, i2):
        wait_gather(i0)
        mask_dst(i0)

        @pl.when(k >= 2)
        def _():
            wait_scatter(i1)
        issue_scatter(i0)

        @pl.when(k + 1 < cnt)
        def _():
            wait_idx(i1)
            issue_gather(i1)

        @pl.when(k + 2 < cnt)
        def _():
            issue_idx(k + 2, i2)

    issue_idx(0, slots[0])
    wait_idx(slots[0])
    issue_gather(slots[0])
    issue_idx(1, slots[1])

    def triple(p, _):
        chunk_step(p * 3, slots[0], slots[1], slots[2])
        chunk_step(p * 3 + 1, slots[1], slots[2], slots[0])
        chunk_step(p * 3 + 2, slots[2], slots[0], slots[1])
        return 0
    lax.fori_loop(0, 130, triple, 0)

    @pl.when(cnt == 391)
    def _():
        chunk_step(390, slots[0], slots[1], slots[2])
        wait_scatter(slots[2])
        wait_scatter(slots[0])

    @pl.when(cnt == 390)
    def _():
        wait_scatter(slots[1])
        wait_scatter(slots[2])

    plsc.subcore_barrier()

    g0 = lo + l0

    @pl.when(s < NS - 1)
    def _():
        pltpu.sync_copy(agg_sh.at[pl.ds(l0, TROWS)], agg_out.at[pl.ds(g0, TROWS)])

    @pl.when(s == NS - 1)
    def _():
        pltpu.sync_copy(agg_sh.at[pl.ds(l0, LTROWS)], agg_out.at[pl.ds(g0, LTROWS)])


@functools.partial(
    pl.kernel,
    out_type=jax.ShapeDtypeStruct((E,), jnp.float32),
    mesh=_MESH,
    compiler_params=pltpu.CompilerParams(use_tc_tiling_on_sc=False, needs_layout_passes=False),
    scratch_types=[
        pltpu.VMEM((CPE,), jnp.int32),
        pltpu.VMEM((CPE,), jnp.int32),
        pltpu.VMEM((CPE,), jnp.int32),
        pltpu.VMEM((CPE,), jnp.int32),
        pltpu.VMEM((CPE, D), jnp.float32),
        pltpu.VMEM((CPE, D), jnp.float32),
        pltpu.VMEM((CPE, D), jnp.float32),
        pltpu.VMEM((CPE, D), jnp.float32),
        pltpu.VMEM((196 * CPE,), jnp.float32),
        pltpu.SemaphoreType.DMA,
        pltpu.SemaphoreType.DMA,
        pltpu.SemaphoreType.DMA,
        pltpu.SemaphoreType.DMA,
        pltpu.SemaphoreType.DMA,
        pltpu.SemaphoreType.DMA,
        pltpu.SemaphoreType.DMA,
        pltpu.SemaphoreType.DMA,
    ],
)
def _rate(ref_hbm, src_hbm, dst_hbm, out_hbm,
          si0, si1, di0, di1, ra0, ra1, rb0, rb1, outbuf,
          ss0, ss1, ds0, ds1, ga0, ga1, gb0, gb1):
    c = lax.axis_index("c")
    s = lax.axis_index("s")
    w = c * NS + s
    start = w * 195 + jnp.minimum(w, 10)
    cnt = jnp.where(w < 10, 196, 195)
    slots = (
        (si0, di0, ra0, rb0, ss0, ds0, ga0, gb0),
        (si1, di1, ra1, rb1, ss1, ds1, ga1, gb1),
    )

    def issue_idx(k, slot):
        base = pl.multiple_of((start + k) * CPE, 8)
        pltpu.async_copy(src_hbm.at[pl.ds(base, CPE)], slot[0], slot[4])
        pltpu.async_copy(dst_hbm.at[pl.ds(base, CPE)], slot[1], slot[5])

    def wait_idx(slot):
        pltpu.make_async_copy(src_hbm.at[pl.ds(0, CPE)], slot[0], slot[4]).wait()
        pltpu.make_async_copy(dst_hbm.at[pl.ds(0, CPE)], slot[1], slot[5]).wait()

    def issue_gather(slot):
        pltpu.async_copy(ref_hbm.at[slot[0]], slot[2], slot[6])
        pltpu.async_copy(ref_hbm.at[slot[1]], slot[3], slot[7])

    def wait_gather(slot):
        pltpu.make_async_copy(ref_hbm.at[slot[0]], slot[2], slot[6]).wait()
        pltpu.make_async_copy(ref_hbm.at[slot[1]], slot[3], slot[7]).wait()

    iota16 = lax.iota(jnp.int32, 16)

    def dots(k, slot):
        ra, rb = slot[2], slot[3]
        ob = k * CPE

        def grp(g, _):
            res = jnp.zeros((16,), jnp.float32)
            for e in range(16):
                row = g * 16 + e
                p = ra[row, pl.ds(0, 16)] * rb[row, pl.ds(0, 16)]
                for u in range(1, D // 16):
                    sl = pl.ds(u * 16, 16)
                    p = p + ra[row, sl] * rb[row, sl]
                res = jnp.where(iota16 == e, jnp.sum(p), res)
            outbuf[pl.ds(ob + g * 16, 16)] = res
            return 0
        lax.fori_loop(0, CPE // 16, grp, 0)

    def chunk_step(k, cur, nxt):
        @pl.when(k + 1 < cnt)
        def _():
            wait_idx(nxt)
            issue_gather(nxt)
        wait_gather(cur)
        dots(k, cur)

        @pl.when(k + 2 < cnt)
        def _():
            issue_idx(k + 2, cur)

    issue_idx(0, slots[0])
    wait_idx(slots[0])
    issue_gather(slots[0])
    issue_idx(1, slots[1])

    def pair(p, _):
        chunk_step(p * 2, slots[0], slots[1])
        chunk_step(p * 2 + 1, slots[1], slots[0])
        return 0
    lax.fori_loop(0, 97, pair, 0)

    chunk_step(194, slots[0], slots[1])

    @pl.when(cnt == 196)
    def _():
        chunk_step(195, slots[1], slots[0])

    obase = pl.multiple_of(start * CPE, 8)

    @pl.when(w < 10)
    def _():
        pltpu.sync_copy(outbuf.at[pl.ds(0, 196 * CPE)],
                        out_hbm.at[pl.ds(obase, 196 * CPE)])

    @pl.when(w >= 10)
    def _():
        pltpu.sync_copy(outbuf.at[pl.ds(0, 195 * CPE)],
                        out_hbm.at[pl.ds(obase, 195 * CPE)])


def kernel(edge_index, edge_attr, movie_w, user_w,
           W1_self, W1_neigh, b1, W2_self, W2_neigh, b2):
    src = edge_index[0]
    dst = edge_index[1]
    x = jnp.concatenate([movie_w, user_w], axis=0)
    deg = _deg(dst)
    y1, z1 = _tc_proj((x,), W1_neigh, W1_self, b1.reshape(1, D), do_relu=False)
    agg1 = _agg(y1, src, dst)
    y2, z2 = _tc_proj((z1, agg1, deg), W2_neigh, W2_self,
                      b2.reshape(1, D), do_relu=True)
    agg2 = _agg(y2, src, dst)
    refined = _tc_combine(z2, agg2, deg)
    ratings = _rate(refined, src, dst)
    return ratings, refined


# speedup vs baseline: 6.9105x; 1.2328x over previous
import functools

import jax
import jax.numpy as jnp
from jax import lax
from jax.experimental import pallas as pl
from jax.experimental.pallas import tpu as pltpu
from jax.experimental.pallas import tpu_sc as plsc

N = 50000
E = 800000
D = 64
NC, NS = 2, 16
HALF = N // 2
TROWS = 1568
SROWS = NS * TROWS
DUMMY = 25080
CPE = 128
NCHUNK = E // CPE
LTROWS = HALF - (NS - 1) * TROWS

_MESH = plsc.VectorSubcoreMesh(core_axis_name="c", subcore_axis_name="s")


def _tc_proj(operands, w_neigh, w_self, b2d, do_relu):
    R = 2000
    with_agg = len(operands) == 3

    def body(*refs):
        if with_agg:
            z_ref, a_ref, d_ref = refs[0:3]
            wn_ref, ws_ref, b_ref, y_ref, zo_ref = refs[3:]
            inv = 1.0 / jnp.maximum(d_ref[...][:, 0:1], 1.0)
            t = z_ref[...] + a_ref[...] * inv
        else:
            (z_ref, wn_ref, ws_ref, b_ref, y_ref, zo_ref) = refs
            t = z_ref[...]
        if do_relu:
            t = jnp.maximum(t, 0.0)
        y_ref[...] = jnp.dot(t, wn_ref[...], preferred_element_type=jnp.float32)
        zo_ref[...] = (
            jnp.dot(t, ws_ref[...], preferred_element_type=jnp.float32) + b_ref[...]
        )

    row_specs = [pl.BlockSpec((R, D), lambda i: (i, 0))] * (2 if with_agg else 1)
    if with_agg:
        row_specs.append(pl.BlockSpec((R, 16), lambda i: (i, 0)))
    return pl.pallas_call(
        body,
        grid=(N // R,),
        in_specs=row_specs + [
            pl.BlockSpec((D, D), lambda i: (0, 0)),
            pl.BlockSpec((D, D), lambda i: (0, 0)),
            pl.BlockSpec((1, D), lambda i: (0, 0)),
        ],
        out_specs=[
            pl.BlockSpec((R, D), lambda i: (i, 0)),
            pl.BlockSpec((R, D), lambda i: (i, 0)),
        ],
        out_shape=[
            jax.ShapeDtypeStruct((N, D), jnp.float32),
            jax.ShapeDtypeStruct((N, D), jnp.float32),
        ],
    )(*operands, w_neigh, w_self, b2d)


def _tc_combine(z, agg, deg):
    R = 2000

    def body(z_ref, a_ref, d_ref, o_ref):
        inv = 1.0 / jnp.maximum(d_ref[...][:, 0:1], 1.0)
        o_ref[...] = z_ref[...] + a_ref[...] * inv

    return pl.pallas_call(
        body,
        grid=(N // R,),
        in_specs=[
            pl.BlockSpec((R, D), lambda i: (i, 0)),
            pl.BlockSpec((R, D), lambda i: (i, 0)),
            pl.BlockSpec((R, 16), lambda i: (i, 0)),
        ],
        out_specs=pl.BlockSpec((R, D), lambda i: (i, 0)),
        out_shape=jax.ShapeDtypeStruct((N, D), jnp.float32),
    )(z, agg, deg)


DSUP = 2000


@functools.partial(
    pl.kernel,
    out_type=jax.ShapeDtypeStruct((N, 16), jnp.float32),
    mesh=_MESH,
    compiler_params=pltpu.CompilerParams(use_tc_tiling_on_sc=False, needs_layout_passes=False),
    scratch_types=[
        pltpu.VMEM((DSUP,), jnp.int32),
        pltpu.VMEM((DSUP,), jnp.int32),
        pltpu.VMEM((TROWS * 16 + 16,), jnp.float32),
        pltpu.VMEM((TROWS, 16), jnp.float32),
        pltpu.SemaphoreType.DMA,
        pltpu.SemaphoreType.DMA,
    ],
)
def _deg(dst_hbm, deg_out, di0, di1, hist, row2d, ds0, ds1):
    c = lax.axis_index("c")
    s = lax.axis_index("s")
    tlo = c * HALF + s * TROWS
    z16 = jnp.zeros((16,), jnp.float32)
    ones16 = jnp.ones((16,), jnp.float32)
    slots = ((di0, ds0), (di1, ds1))
    nsup = E // (NS * DSUP)

    def _zh(i, _):
        hist[pl.ds(i * 16, 16)] = z16
        return 0
    lax.fori_loop(0, TROWS + 1, _zh, 0)

    def issue_idx(q, slot):
        base = pl.multiple_of((s * nsup + q) * DSUP, 8)
        pltpu.async_copy(dst_hbm.at[pl.ds(base, DSUP)], slot[0], slot[1])

    def wait_idx(slot):
        pltpu.make_async_copy(dst_hbm.at[pl.ds(0, DSUP)], slot[0], slot[1]).wait()

    def scan(slot):
        di = slot[0]

        def body(v, _):
            d = di[pl.ds(v * 16, 16)]
            lr = d - tlo
            ok = jnp.logical_and(lr >= 0, lr < TROWS)
            idx = jnp.where(ok, lr * 16, TROWS * 16)
            plsc.addupdate_scatter(hist, [idx], ones16)
            return 0
        lax.fori_loop(0, DSUP // 16, body, 0)

    issue_idx(0, slots[0])
    wait_idx(slots[0])
    issue_idx(1, slots[1])

    def pair(p, _):
        scan(slots[0])

        @pl.when(p * 2 + 2 < nsup)
        def _():
            issue_idx(p * 2 + 2, slots[0])

        @pl.when(p * 2 + 1 < nsup)
        def _():
            wait_idx(slots[1])
        scan(slots[1])

        @pl.when(p * 2 + 3 < nsup)
        def _():
            issue_idx(p * 2 + 3, slots[1])

        @pl.when(p * 2 + 2 < nsup)
        def _():
            wait_idx(slots[0])
        return 0
    lax.fori_loop(0, nsup // 2, pair, 0)

    scan(slots[0])

    def stage(r, _):
        row2d[r, pl.ds(0, 16)] = hist[pl.ds(r * 16, 16)]
        return 0
    lax.fori_loop(0, TROWS, stage, 0)

    @pl.when(s < NS - 1)
    def _():
        pltpu.sync_copy(row2d, deg_out.at[pl.ds(tlo, TROWS)])

    @pl.when(s == NS - 1)
    def _():
        pltpu.sync_copy(row2d.at[pl.ds(0, LTROWS)], deg_out.at[pl.ds(tlo, LTROWS)])


@functools.partial(
    pl.kernel,
    out_type=jax.ShapeDtypeStruct((N, D), jnp.float32),
    mesh=_MESH,
    compiler_params=pltpu.CompilerParams(use_tc_tiling_on_sc=False, needs_layout_passes=False),
    scratch_types=[
        pltpu.VMEM((CPE,), jnp.int32),
        pltpu.VMEM((CPE,), jnp.int32),
        pltpu.VMEM((CPE,), jnp.int32),
        pltpu.VMEM((CPE,), jnp.int32),
        pltpu.VMEM((CPE,), jnp.int32),
        pltpu.VMEM((CPE,), jnp.int32),
        pltpu.VMEM((CPE,), jnp.int32),
        pltpu.VMEM((CPE,), jnp.int32),
        pltpu.VMEM((CPE,), jnp.int32),
        pltpu.VMEM((CPE, D), jnp.float32),
        pltpu.VMEM((CPE, D), jnp.float32),
        pltpu.VMEM((CPE, D), jnp.float32),
        pltpu.VMEM((56, D), jnp.float32),
        pltpu.VMEM_SHARED((SROWS, D), jnp.float32),
        pltpu.SemaphoreType.DMA,
        pltpu.SemaphoreType.DMA,
        pltpu.SemaphoreType.DMA,
        pltpu.SemaphoreType.DMA,
        pltpu.SemaphoreType.DMA,
        pltpu.SemaphoreType.DMA,
        pltpu.SemaphoreType.DMA,
        pltpu.SemaphoreType.DMA,
        pltpu.SemaphoreType.DMA,
    ],
)
def _agg(y_hbm, src_hbm, dst_hbm, agg_out,
         si0, si1, si2, di0, di1, di2, dm0, dm1, dm2, r0, r1, r2, zbuf,
         agg_sh, is0, is1, is2, gs0, gs1, gs2, sc0, sc1, sc2):
    c = lax.axis_index("c")
    s = lax.axis_index("s")
    lo = c * HALF
    start = s * 390 + jnp.minimum(s, 10)
    cnt = jnp.where(s < 10, 391, 390)
    slots = (
        (si0, di0, dm0, r0, is0, gs0, sc0),
        (si1, di1, dm1, r1, is1, gs1, sc1),
        (si2, di2, dm2, r2, is2, gs2, sc2),
    )
    z16 = jnp.zeros((16,), jnp.float32)
    l0 = pl.multiple_of(s * TROWS, 8)

    def _zb(r, _):
        for u in range(D // 16):
            zbuf[r, pl.ds(u * 16, 16)] = z16
        return 0
    lax.fori_loop(0, 56, _zb, 0)

    for q in range(TROWS // 56):
        pltpu.sync_copy(zbuf, agg_sh.at[pl.ds(l0 + q * 56, 56)])
    plsc.subcore_barrier()

    def issue_idx(k, slot):
        base = pl.multiple_of((start + k) * CPE, 8)
        pltpu.async_copy(src_hbm.at[pl.ds(base, CPE)], slot[0], slot[4])
        pltpu.async_copy(dst_hbm.at[pl.ds(base, CPE)], slot[1], slot[4])

    def wait_idx(slot):
        pltpu.make_async_copy(src_hbm.at[pl.ds(0, CPE)], slot[0], slot[4]).wait()
        pltpu.make_async_copy(dst_hbm.at[pl.ds(0, CPE)], slot[1], slot[4]).wait()

    def issue_gather(slot):
        pltpu.async_copy(y_hbm.at[slot[0]], slot[3], slot[5])

    def wait_gather(slot):
        pltpu.make_async_copy(y_hbm.at[slot[0]], slot[3], slot[5]).wait()

    def issue_scatter(slot):
        pltpu.async_copy(slot[3], agg_sh.at[slot[2]], slot[6], add=True)

    def wait_scatter(slot):
        pltpu.make_async_copy(slot[3], agg_sh.at[slot[2]], slot[6]).wait()

    def mask_dst(slot):
        di, dm = slot[1], slot[2]
        for v in range(CPE // 16):
            d = di[pl.ds(v * 16, 16)]
            rloc = d - lo
            ok = jnp.logical_and(rloc >= 0, rloc < HALF)
            dm[pl.ds(v * 16, 16)] = jnp.where(ok, rloc, DUMMY)

    def chunk_step(k, i0, i1, i2):
        wait_gather(i0)
        mask_dst(i0)

        @pl.when(k >= 2)
        def _():
            wait_scatter(i1)
        issue_scatter(i0)

        @pl.when(k + 1 < cnt)
        def _():
            wait_idx(i1)
            issue_gather(i1)

        @pl.when(k + 2 < cnt)
        def _():
            issue_idx(k + 2, i2)

    issue_idx(0, slots[0])
    wait_idx(slots[0])
    issue_gather(slots[0])
    issue_idx(1, slots[1])

    def triple(p, _):
        chunk_step(p * 3, slots[0], slots[1], slots[2])
        chunk_step(p * 3 + 1, slots[1], slots[2], slots[0])
        chunk_step(p * 3 + 2, slots[2], slots[0], slots[1])
        return 0
    lax.fori_loop(0, 130, triple, 0)

    @pl.when(cnt == 391)
    def _():
        chunk_step(390, slots[0], slots[1], slots[2])
        wait_scatter(slots[2])
        wait_scatter(slots[0])

    @pl.when(cnt == 390)
    def _():
        wait_scatter(slots[1])
        wait_scatter(slots[2])

    plsc.subcore_barrier()

    g0 = lo + l0

    @pl.when(s < NS - 1)
    def _():
        pltpu.sync_copy(agg_sh.at[pl.ds(l0, TROWS)], agg_out.at[pl.ds(g0, TROWS)])

    @pl.when(s == NS - 1)
    def _():
        pltpu.sync_copy(agg_sh.at[pl.ds(l0, LTROWS)], agg_out.at[pl.ds(g0, LTROWS)])


@functools.partial(
    pl.kernel,
    out_type=jax.ShapeDtypeStruct((E,), jnp.float32),
    mesh=_MESH,
    compiler_params=pltpu.CompilerParams(use_tc_tiling_on_sc=False, needs_layout_passes=False),
    scratch_types=[
        pltpu.VMEM((CPE,), jnp.int32),
        pltpu.VMEM((CPE,), jnp.int32),
        pltpu.VMEM((CPE,), jnp.int32),
        pltpu.VMEM((CPE,), jnp.int32),
        pltpu.VMEM((CPE, D), jnp.float32),
        pltpu.VMEM((CPE, D), jnp.float32),
        pltpu.VMEM((CPE, D), jnp.float32),
        pltpu.VMEM((CPE, D), jnp.float32),
        pltpu.VMEM((196 * CPE,), jnp.float32),
        pltpu.SemaphoreType.DMA,
        pltpu.SemaphoreType.DMA,
        pltpu.SemaphoreType.DMA,
        pltpu.SemaphoreType.DMA,
        pltpu.SemaphoreType.DMA,
        pltpu.SemaphoreType.DMA,
        pltpu.SemaphoreType.DMA,
        pltpu.SemaphoreType.DMA,
    ],
)
def _rate(ref_hbm, src_hbm, dst_hbm, out_hbm,
          si0, si1, di0, di1, ra0, ra1, rb0, rb1, outbuf,
          ss0, ss1, ds0, ds1, ga0, ga1, gb0, gb1):
    c = lax.axis_index("c")
    s = lax.axis_index("s")
    w = c * NS + s
    start = w * 195 + jnp.minimum(w, 10)
    cnt = jnp.where(w < 10, 196, 195)
    slots = (
        (si0, di0, ra0, rb0, ss0, ds0, ga0, gb0),
        (si1, di1, ra1, rb1, ss1, ds1, ga1, gb1),
    )

    def issue_idx(k, slot):
        base = pl.multiple_of((start + k) * CPE, 8)
        pltpu.async_copy(src_hbm.at[pl.ds(base, CPE)], slot[0], slot[4])
        pltpu.async_copy(dst_hbm.at[pl.ds(base, CPE)], slot[1], slot[5])

    def wait_idx(slot):
        pltpu.make_async_copy(src_hbm.at[pl.ds(0, CPE)], slot[0], slot[4]).wait()
        pltpu.make_async_copy(dst_hbm.at[pl.ds(0, CPE)], slot[1], slot[5]).wait()

    def issue_gather(slot):
        pltpu.async_copy(ref_hbm.at[slot[0]], slot[2], slot[6])
        pltpu.async_copy(ref_hbm.at[slot[1]], slot[3], slot[7])

    def wait_gather(slot):
        pltpu.make_async_copy(ref_hbm.at[slot[0]], slot[2], slot[6]).wait()
        pltpu.make_async_copy(ref_hbm.at[slot[1]], slot[3], slot[7]).wait()

    iota16 = lax.iota(jnp.int32, 16)

    def dots(k, slot):
        ra, rb = slot[2], slot[3]
        ob = k * CPE

        def grp(g, _):
            res = jnp.zeros((16,), jnp.float32)
            for e in range(16):
                row = g * 16 + e
                p = ra[row, pl.ds(0, 16)] * rb[row, pl.ds(0, 16)]
                for u in range(1, D // 16):
                    sl = pl.ds(u * 16, 16)
                    p = p + ra[row, sl] * rb[row, sl]
                res = jnp.where(iota16 == e, jnp.sum(p), res)
            outbuf[pl.ds(ob + g * 16, 16)] = res
            return 0
        lax.fori_loop(0, CPE // 16, grp, 0)

    def chunk_step(k, cur, nxt):
        @pl.when(k + 1 < cnt)
        def _():
            wait_idx(nxt)
            issue_gather(nxt)
        wait_gather(cur)
        dots(k, cur)

        @pl.when(k + 2 < cnt)
        def _():
            issue_idx(k + 2, cur)

    issue_idx(0, slots[0])
    wait_idx(slots[0])
    issue_gather(slots[0])
    issue_idx(1, slots[1])

    def pair(p, _):
        chunk_step(p * 2, slots[0], slots[1])
        chunk_step(p * 2 + 1, slots[1], slots[0])
        return 0
    lax.fori_loop(0, 97, pair, 0)

    chunk_step(194, slots[0], slots[1])

    @pl.when(cnt == 196)
    def _():
        chunk_step(195, slots[1], slots[0])

    obase = pl.multiple_of(start * CPE, 8)

    @pl.when(w < 10)
    def _():
        pltpu.sync_copy(outbuf.at[pl.ds(0, 196 * CPE)],
                        out_hbm.at[pl.ds(obase, 196 * CPE)])

    @pl.when(w >= 10)
    def _():
        pltpu.sync_copy(outbuf.at[pl.ds(0, 195 * CPE)],
                        out_hbm.at[pl.ds(obase, 195 * CPE)])


def kernel(edge_index, edge_attr, movie_w, user_w,
           W1_self, W1_neigh, b1, W2_self, W2_neigh, b2):
    src = edge_index[0]
    dst = edge_index[1]
    x = jnp.concatenate([movie_w, user_w], axis=0)
    deg = _deg(dst)
    y1, z1 = _tc_proj((x,), W1_neigh, W1_self, b1.reshape(1, D), do_relu=False)
    agg1 = _agg(y1, src, dst)
    y2, z2 = _tc_proj((z1, agg1, deg), W2_neigh, W2_self,
                      b2.reshape(1, D), do_relu=True)
    agg2 = _agg(y2, src, dst)
    refined = _tc_combine(z2, agg2, deg)
    ratings = _rate(refined, src, dst)
    return ratings, refined
